# baseline (speedup 1.0000x reference)
import functools

import jax
import jax.numpy as jnp
from jax import lax
from jax.experimental import pallas as pl
from jax.experimental.pallas import tpu as pltpu
from jax.experimental.pallas import tpu_sc as plsc

N = 10000
NP = 10240
E = 160000
D = 128
ED = 16
H = 1024
G = 50
GP = 64

NCH = 5
CH = E // NCH
NC, NS = 2, 16
NW = NC * NS
EPW = CH // NW
GCH = 200
TE = 2000
TN = 640

_MESH_KW = dict(core_axis_name="c", subcore_axis_name="s",
                num_cores=NC, num_subcores=NS)


@functools.lru_cache(maxsize=None)
def _build_sc_gather():
    mesh = plsc.VectorSubcoreMesh(**_MESH_KW)

    @functools.partial(
        pl.kernel,
        out_type=(jax.ShapeDtypeStruct((CH, D), jnp.float32),
                  jax.ShapeDtypeStruct((CH, D), jnp.float32)),
        mesh=mesh,
        scratch_types=[
            pltpu.VMEM((GCH,), jnp.int32),
            pltpu.VMEM((GCH, D), jnp.float32),
            pltpu.SemaphoreType.DMA,
        ],
    )
    def sc_gather(h_hbm, src_hbm, dst_hbm, hs_out, hd_out, idx_v, rows_v, sem):
        wid = lax.axis_index("c") * NS + lax.axis_index("s")
        base = wid * EPW
        for idx_hbm, out_hbm in ((src_hbm, hs_out), (dst_hbm, hd_out)):
            def body(j, _, idx_hbm=idx_hbm, out_hbm=out_hbm):
                off = base + j * GCH
                pltpu.sync_copy(idx_hbm.at[pl.ds(off, GCH)], idx_v)
                pltpu.async_copy(h_hbm.at[idx_v], rows_v, sem).wait()
                pltpu.sync_copy(rows_v, out_hbm.at[pl.ds(off, GCH)])
                return 0
            lax.fori_loop(0, EPW // GCH, body, 0)

    return sc_gather


@functools.lru_cache(maxsize=None)
def _build_sc_scatter():
    mesh = plsc.VectorSubcoreMesh(**_MESH_KW)

    @functools.partial(
        pl.kernel,
        out_type=jax.ShapeDtypeStruct((NC, NP, ED), jnp.float32),
        mesh=mesh,
        scratch_types=[
            pltpu.VMEM((EPW,), jnp.int32),
            pltpu.VMEM((EPW, ED), jnp.float32),
            pltpu.VMEM_SHARED((NP, ED), jnp.float32),
        ],
        compiler_params=pltpu.CompilerParams(use_tc_tiling_on_sc=False),
    )
    def sc_scatter(eupd_hbm, dst_hbm, zeros_hbm, out_hbm, idx_v, rows_v,
                   acc_sh):
        c = lax.axis_index("c")
        s = lax.axis_index("s")
        rows_per_sub = NP // NS
        pltpu.sync_copy(zeros_hbm.at[pl.ds(s * rows_per_sub, rows_per_sub)],
                        acc_sh.at[pl.ds(s * rows_per_sub, rows_per_sub)])
        plsc.subcore_barrier()
        base = (c * NS + s) * EPW
        pltpu.sync_copy(dst_hbm.at[pl.ds(base, EPW)], idx_v)
        pltpu.sync_copy(eupd_hbm.at[pl.ds(base, EPW)], rows_v)
        pltpu.sync_copy(rows_v, acc_sh.at[idx_v], add=True)
        plsc.subcore_barrier()
        pltpu.sync_copy(acc_sh.at[pl.ds(s * rows_per_sub, rows_per_sub)],
                        out_hbm.at[c, pl.ds(s * rows_per_sub, rows_per_sub)])

    return sc_scatter


def _edge_body(hs_ref, hd_ref, e_ref, wsd_ref, we_ref, b1_ref,
               w2_ref, b2_ref, eupd_ref, enew_ref):
    hsd = jnp.concatenate([hs_ref[...], hd_ref[...]],
                          axis=1).astype(jnp.bfloat16)
    e_bf = e_ref[...].astype(jnp.bfloat16)
    dn = (((1,), (1,)), ((), ()))
    f32 = jnp.float32
    hid = (lax.dot_general(hsd, wsd_ref[...], dn, preferred_element_type=f32)
           + lax.dot_general(e_bf, we_ref[...], dn, preferred_element_type=f32))
    hid = jnp.maximum(hid + b1_ref[...], 0.0).astype(jnp.bfloat16)
    eupd = lax.dot_general(hid, w2_ref[...], dn,
                           preferred_element_type=f32) + b2_ref[...]
    eupd_ref[...] = eupd
    enew_ref[...] = e_ref[...] + eupd


def _edge_mlp(hs, hd, e, p):
    grid = (CH // TE,)
    w1_bf = p['eW1'].astype(jnp.bfloat16)
    w_sd = jnp.concatenate([w1_bf[:, :D], w1_bf[:, D + ED:]], axis=1)
    return pl.pallas_call(
        _edge_body,
        grid=grid,
        in_specs=[
            pl.BlockSpec((TE, D), lambda i: (i, 0)),
            pl.BlockSpec((TE, D), lambda i: (i, 0)),
            pl.BlockSpec((TE, ED), lambda i: (i, 0)),
            pl.BlockSpec((H, 2 * D), lambda i: (0, 0)),
            pl.BlockSpec((H, ED), lambda i: (0, 0)),
            pl.BlockSpec((1, H), lambda i: (0, 0)),
            pl.BlockSpec((ED, H), lambda i: (0, 0)),
            pl.BlockSpec((1, ED), lambda i: (0, 0)),
        ],
        out_specs=[
            pl.BlockSpec((TE, ED), lambda i: (i, 0)),
            pl.BlockSpec((TE, ED), lambda i: (i, 0)),
        ],
        out_shape=[
            jax.ShapeDtypeStruct((CH, ED), jnp.float32),
            jax.ShapeDtypeStruct((CH, ED), jnp.float32),
        ],
        compiler_params=pltpu.CompilerParams(
            dimension_semantics=("arbitrary",)),
    )(hs, hd, e, w_sd, w1_bf[:, D:D + ED],
      p['eb1'].reshape(1, H), p['eW2'].astype(jnp.bfloat16),
      p['eb2'].reshape(1, ED))


def _node_body(h_ref, *rest):
    parts_refs = rest[:NCH]
    w1_ref, b1_ref, w2_ref, b2_ref, out_ref = rest[NCH:]
    h = h_ref[...]
    esum = parts_refs[0][0] + parts_refs[0][1]
    for pr in parts_refs[1:]:
        esum = esum + pr[0] + pr[1]
    nin = jnp.concatenate([h, esum], axis=1).astype(jnp.bfloat16)
    hid = lax.dot_general(nin, w1_ref[...], (((1,), (1,)), ((), ())),
                          preferred_element_type=jnp.float32)
    hid = jnp.maximum(hid + b1_ref[...], 0.0).astype(jnp.bfloat16)
    upd = lax.dot_general(hid, w2_ref[...], (((1,), (1,)), ((), ())),
                          preferred_element_type=jnp.float32) + b2_ref[...]
    out_ref[...] = h + upd


def _node_mlp(h, parts_list, p):
    grid = (NP // TN,)
    parts_spec = pl.BlockSpec((NC, TN, ED), lambda i: (0, i, 0))
    return pl.pallas_call(
        _node_body,
        grid=grid,
        in_specs=[
            pl.BlockSpec((TN, D), lambda i: (i, 0)),
            *([parts_spec] * NCH),
            pl.BlockSpec((H, D + ED), lambda i: (0, 0)),
            pl.BlockSpec((1, H), lambda i: (0, 0)),
            pl.BlockSpec((D, H), lambda i: (0, 0)),
            pl.BlockSpec((1, D), lambda i: (0, 0)),
        ],
        out_specs=pl.BlockSpec((TN, D), lambda i: (i, 0)),
        out_shape=jax.ShapeDtypeStruct((NP, D), jnp.float32),
        compiler_params=pltpu.CompilerParams(
            dimension_semantics=("arbitrary",)),
    )(h, *parts_list, p['nW1'].astype(jnp.bfloat16), p['nb1'].reshape(1, H),
      p['nW2'].astype(jnp.bfloat16), p['nb2'].reshape(1, D))


def _gru_body(h_ref, gid_ref, wih_ref, bih_ref, bhh_ref, out_ref):
    h = h_ref[...].astype(jnp.bfloat16)
    gi = lax.dot_general(h, wih_ref[...], (((1,), (1,)), ((), ())),
                         preferred_element_type=jnp.float32) + bih_ref[...]
    i_r = gi[:, :H]
    i_z = gi[:, H:2 * H]
    i_n = gi[:, 2 * H:]
    bhh = bhh_ref[...]
    r = jax.nn.sigmoid(i_r + bhh[:, :H])
    z = jax.nn.sigmoid(i_z + bhh[:, H:2 * H])
    n = jnp.tanh(i_n + r * bhh[:, 2 * H:])
    feat = (1.0 - z) * n
    feat = jax.nn.sigmoid(feat) * feat
    ids = gid_ref[0, 0, :]
    onehot = (ids[:, None] == lax.broadcasted_iota(jnp.int32, (TN, GP), 1)
              ).astype(jnp.float32)
    contrib = lax.dot_general(onehot, feat, (((0,), (0,)), ((), ())),
                              preferred_element_type=jnp.float32)
    @pl.when(pl.program_id(0) == 0)
    def _():
        out_ref[...] = jnp.zeros_like(out_ref)
    out_ref[...] += contrib


def _gru_readout(h, gids, gp):
    grid = (NP // TN,)
    return pl.pallas_call(
        _gru_body,
        grid=grid,
        in_specs=[
            pl.BlockSpec((TN, D), lambda i: (i, 0)),
            pl.BlockSpec((1, 1, TN), lambda i: (i, 0, 0)),
            pl.BlockSpec((3 * H, D), lambda i: (0, 0)),
            pl.BlockSpec((1, 3 * H), lambda i: (0, 0)),
            pl.BlockSpec((1, 3 * H), lambda i: (0, 0)),
        ],
        out_specs=pl.BlockSpec((GP, H), lambda i: (0, 0)),
        out_shape=jax.ShapeDtypeStruct((GP, H), jnp.float32),
        compiler_params=pltpu.CompilerParams(
            dimension_semantics=("arbitrary",)),
    )(h, gids, gp['W_ih'].astype(jnp.bfloat16), gp['b_ih'].reshape(1, 3 * H),
      gp['b_hh'].reshape(1, 3 * H))


def kernel(x, edge_index, edge_attr, graph_ids, params):
    src = edge_index[0].astype(jnp.int32)
    dst = edge_index[1].astype(jnp.int32)
    h = jnp.zeros((NP, D), jnp.float32).at[:N].set(x)
    src_ch = [src[k * CH:(k + 1) * CH] for k in range(NCH)]
    dst_ch = [dst[k * CH:(k + 1) * CH] for k in range(NCH)]
    e_ch = [edge_attr[k * CH:(k + 1) * CH] for k in range(NCH)]
    zeros16 = jnp.zeros((NP, ED), jnp.float32)
    gids = jnp.concatenate(
        [graph_ids.astype(jnp.int32),
         jnp.full((NP - N,), GP - 1, jnp.int32)]).reshape(NP // TN, 1, TN)
    gather = _build_sc_gather()
    scatter = _build_sc_scatter()
    for i in range(3):
        p = params['l%d' % i]
        gathered = [gather(h, src_ch[k], dst_ch[k]) for k in range(NCH)]
        edged = [_edge_mlp(gathered[k][0], gathered[k][1], e_ch[k], p)
                 for k in range(NCH)]
        parts_list = [scatter(edged[k][0], dst_ch[k], zeros16)
                      for k in range(NCH)]
        h = _node_mlp(h, parts_list, p)
        e_ch = [ek[1] for ek in edged]
    out = _gru_readout(h, gids, params['gru'])
    return out[:G]


# speedup vs baseline: 1.1868x; 1.1868x over previous
import functools

import jax
import jax.numpy as jnp
from jax import lax
from jax.experimental import pallas as pl
from jax.experimental.pallas import tpu as pltpu
from jax.experimental.pallas import tpu_sc as plsc

N = 10000
NP = 10240
E = 160000
D = 128
ED = 16
H = 1024
G = 50
GP = 64

NCH = 5
CH = E // NCH
NC, NS = 2, 16
NW = NC * NS
EPW = CH // NW
GCH = 200
SCH = 200
TE = 2000
TN = 640

_MESH_KW = dict(core_axis_name="c", subcore_axis_name="s",
                num_cores=NC, num_subcores=NS)


@functools.lru_cache(maxsize=None)
def _build_sc_gather():
    mesh = plsc.VectorSubcoreMesh(**_MESH_KW)

    @functools.partial(
        pl.kernel,
        out_type=(jax.ShapeDtypeStruct((CH, D), jnp.float32),
                  jax.ShapeDtypeStruct((CH, D), jnp.float32)),
        mesh=mesh,
        scratch_types=[
            pltpu.VMEM((GCH,), jnp.int32),
            pltpu.VMEM((GCH, D), jnp.float32),
            pltpu.SemaphoreType.DMA,
        ],
    )
    def sc_gather(h_hbm, src_hbm, dst_hbm, hs_out, hd_out, idx_v, rows_v, sem):
        wid = lax.axis_index("c") * NS + lax.axis_index("s")
        base = wid * EPW
        for idx_hbm, out_hbm in ((src_hbm, hs_out), (dst_hbm, hd_out)):
            def body(j, _, idx_hbm=idx_hbm, out_hbm=out_hbm):
                off = base + j * GCH
                pltpu.sync_copy(idx_hbm.at[pl.ds(off, GCH)], idx_v)
                pltpu.async_copy(h_hbm.at[idx_v], rows_v, sem).wait()
                pltpu.sync_copy(rows_v, out_hbm.at[pl.ds(off, GCH)])
                return 0
            lax.fori_loop(0, EPW // GCH, body, 0)

    return sc_gather


@functools.lru_cache(maxsize=None)
def _build_sc_scatter():
    mesh = plsc.VectorSubcoreMesh(**_MESH_KW)

    @functools.partial(
        pl.kernel,
        out_type=jax.ShapeDtypeStruct((NC, NP, D), jnp.float32),
        mesh=mesh,
        scratch_types=[
            pltpu.VMEM((EPW,), jnp.int32),
            pltpu.VMEM((EPW, ED), jnp.float32),
            pltpu.VMEM_SHARED((NP, ED), jnp.float32),
        ],
        compiler_params=pltpu.CompilerParams(use_tc_tiling_on_sc=False),
    )
    def sc_scatter(eupd_hbm, dst_hbm, zeros_hbm, out_hbm, idx_v, rows_v,
                   acc_sh):
        c = lax.axis_index("c")
        s = lax.axis_index("s")
        rows_per_sub = NP // NS
        rsub = pl.ds(s * rows_per_sub, rows_per_sub)
        pltpu.sync_copy(zeros_hbm.at[rsub, pl.ds(0, ED)], acc_sh.at[rsub])
        plsc.subcore_barrier()
        base = (c * NS + s) * EPW
        pltpu.sync_copy(dst_hbm.at[pl.ds(base, EPW)], idx_v)
        pltpu.sync_copy(eupd_hbm.at[pl.ds(base, EPW), pl.ds(0, ED)], rows_v)
        pltpu.sync_copy(rows_v, acc_sh.at[idx_v], add=True)
        plsc.subcore_barrier()
        pltpu.sync_copy(acc_sh.at[rsub], out_hbm.at[c, rsub, pl.ds(0, ED)])

    return sc_scatter


def _edge_body(hs_ref, hd_ref, e_ref, wsd_ref, we_ref, b1_ref,
               w2_ref, b2_ref, eupd_ref, enew_ref):
    hsd = jnp.concatenate([hs_ref[...], hd_ref[...]],
                          axis=1).astype(jnp.bfloat16)
    e_bf = e_ref[...].astype(jnp.bfloat16)
    dn = (((1,), (1,)), ((), ()))
    f32 = jnp.float32
    hid = (lax.dot_general(hsd, wsd_ref[...], dn, preferred_element_type=f32)
           + lax.dot_general(e_bf, we_ref[...], dn, preferred_element_type=f32))
    hid = jnp.maximum(hid + b1_ref[...], 0.0).astype(jnp.bfloat16)
    eupd = lax.dot_general(hid, w2_ref[...], dn,
                           preferred_element_type=f32) + b2_ref[...]
    eupd_ref[...] = eupd
    enew_ref[...] = e_ref[...] + eupd[:, :ED]


def _edge_mlp(hs, hd, e, p):
    grid = (CH // TE,)
    w1_bf = p['eW1'].astype(jnp.bfloat16)
    w_sd = jnp.concatenate([w1_bf[:, :D], w1_bf[:, D + ED:]], axis=1)
    return pl.pallas_call(
        _edge_body,
        grid=grid,
        in_specs=[
            pl.BlockSpec((TE, D), lambda i: (i, 0)),
            pl.BlockSpec((TE, D), lambda i: (i, 0)),
            pl.BlockSpec((TE, ED), lambda i: (i, 0)),
            pl.BlockSpec((H, 2 * D), lambda i: (0, 0)),
            pl.BlockSpec((H, ED), lambda i: (0, 0)),
            pl.BlockSpec((1, H), lambda i: (0, 0)),
            pl.BlockSpec((D, H), lambda i: (0, 0)),
            pl.BlockSpec((1, D), lambda i: (0, 0)),
        ],
        out_specs=[
            pl.BlockSpec((TE, D), lambda i: (i, 0)),
            pl.BlockSpec((TE, ED), lambda i: (i, 0)),
        ],
        out_shape=[
            jax.ShapeDtypeStruct((CH, D), jnp.float32),
            jax.ShapeDtypeStruct((CH, ED), jnp.float32),
        ],
        compiler_params=pltpu.CompilerParams(
            dimension_semantics=("arbitrary",)),
    )(hs, hd, e, w_sd, w1_bf[:, D:D + ED],
      p['eb1'].reshape(1, H),
      jnp.zeros((D, H), jnp.bfloat16).at[:ED].set(
          p['eW2'].astype(jnp.bfloat16)),
      jnp.zeros((1, D), jnp.float32).at[0, :ED].set(p['eb2']))


def _node_body(h_ref, *rest):
    parts_refs = rest[:NCH]
    w1_ref, b1_ref, w2_ref, b2_ref, out_ref = rest[NCH:]
    h = h_ref[...]
    esum = parts_refs[0][0, :, :ED] + parts_refs[0][1, :, :ED]
    for pr in parts_refs[1:]:
        esum = esum + pr[0, :, :ED] + pr[1, :, :ED]
    nin = jnp.concatenate([h, esum], axis=1).astype(jnp.bfloat16)
    hid = lax.dot_general(nin, w1_ref[...], (((1,), (1,)), ((), ())),
                          preferred_element_type=jnp.float32)
    hid = jnp.maximum(hid + b1_ref[...], 0.0).astype(jnp.bfloat16)
    upd = lax.dot_general(hid, w2_ref[...], (((1,), (1,)), ((), ())),
                          preferred_element_type=jnp.float32) + b2_ref[...]
    out_ref[...] = h + upd


def _node_mlp(h, parts_list, p):
    grid = (NP // TN,)
    parts_spec = pl.BlockSpec((NC, TN, D), lambda i: (0, i, 0))
    return pl.pallas_call(
        _node_body,
        grid=grid,
        in_specs=[
            pl.BlockSpec((TN, D), lambda i: (i, 0)),
            *([parts_spec] * NCH),
            pl.BlockSpec((H, D + ED), lambda i: (0, 0)),
            pl.BlockSpec((1, H), lambda i: (0, 0)),
            pl.BlockSpec((D, H), lambda i: (0, 0)),
            pl.BlockSpec((1, D), lambda i: (0, 0)),
        ],
        out_specs=pl.BlockSpec((TN, D), lambda i: (i, 0)),
        out_shape=jax.ShapeDtypeStruct((NP, D), jnp.float32),
        compiler_params=pltpu.CompilerParams(
            dimension_semantics=("arbitrary",)),
    )(h, *parts_list, p['nW1'].astype(jnp.bfloat16), p['nb1'].reshape(1, H),
      p['nW2'].astype(jnp.bfloat16), p['nb2'].reshape(1, D))


def _gru_body(h_ref, gid_ref, wih_ref, bih_ref, bhh_ref, out_ref):
    h = h_ref[...].astype(jnp.bfloat16)
    gi = lax.dot_general(h, wih_ref[...], (((1,), (1,)), ((), ())),
                         preferred_element_type=jnp.float32) + bih_ref[...]
    i_r = gi[:, :H]
    i_z = gi[:, H:2 * H]
    i_n = gi[:, 2 * H:]
    bhh = bhh_ref[...]
    r = jax.nn.sigmoid(i_r + bhh[:, :H])
    z = jax.nn.sigmoid(i_z + bhh[:, H:2 * H])
    n = jnp.tanh(i_n + r * bhh[:, 2 * H:])
    feat = (1.0 - z) * n
    feat = jax.nn.sigmoid(feat) * feat
    ids = gid_ref[0, 0, :]
    onehot = (ids[:, None] == lax.broadcasted_iota(jnp.int32, (TN, GP), 1)
              ).astype(jnp.float32)
    contrib = lax.dot_general(onehot, feat, (((0,), (0,)), ((), ())),
                              preferred_element_type=jnp.float32)
    @pl.when(pl.program_id(0) == 0)
    def _():
        out_ref[...] = jnp.zeros_like(out_ref)
    out_ref[...] += contrib


def _gru_readout(h, gids, gp):
    grid = (NP // TN,)
    return pl.pallas_call(
        _gru_body,
        grid=grid,
        in_specs=[
            pl.BlockSpec((TN, D), lambda i: (i, 0)),
            pl.BlockSpec((1, 1, TN), lambda i: (i, 0, 0)),
            pl.BlockSpec((3 * H, D), lambda i: (0, 0)),
            pl.BlockSpec((1, 3 * H), lambda i: (0, 0)),
            pl.BlockSpec((1, 3 * H), lambda i: (0, 0)),
        ],
        out_specs=pl.BlockSpec((GP, H), lambda i: (0, 0)),
        out_shape=jax.ShapeDtypeStruct((GP, H), jnp.float32),
        compiler_params=pltpu.CompilerParams(
            dimension_semantics=("arbitrary",)),
    )(h, gids, gp['W_ih'].astype(jnp.bfloat16), gp['b_ih'].reshape(1, 3 * H),
      gp['b_hh'].reshape(1, 3 * H))


def kernel(x, edge_index, edge_attr, graph_ids, params):
    src = edge_index[0].astype(jnp.int32)
    dst = edge_index[1].astype(jnp.int32)
    h = jnp.zeros((NP, D), jnp.float32).at[:N].set(x)
    src_ch = [src[k * CH:(k + 1) * CH] for k in range(NCH)]
    dst_ch = [dst[k * CH:(k + 1) * CH] for k in range(NCH)]
    e_ch = [edge_attr[k * CH:(k + 1) * CH] for k in range(NCH)]
    zeros16 = jnp.zeros((NP, D), jnp.float32)
    gids = jnp.concatenate(
        [graph_ids.astype(jnp.int32),
         jnp.full((NP - N,), GP - 1, jnp.int32)]).reshape(NP // TN, 1, TN)
    gather = _build_sc_gather()
    scatter = _build_sc_scatter()
    for i in range(3):
        p = params['l%d' % i]
        gathered = [gather(h, src_ch[k], dst_ch[k]) for k in range(NCH)]
        edged = [_edge_mlp(gathered[k][0], gathered[k][1], e_ch[k], p)
                 for k in range(NCH)]
        parts_list = [scatter(edged[k][0], dst_ch[k], zeros16)
                      for k in range(NCH)]
        h = _node_mlp(h, parts_list, p)
        e_ch = [ek[1] for ek in edged]
    out = _gru_readout(h, gids, params['gru'])
    return out[:G]
